# Initial kernel scaffold; baseline (speedup 1.0000x reference)
import jax
import jax.numpy as jnp
from jax.experimental import pallas as pl


def kernel(word, table):
    raise NotImplementedError("write your pallas kernel here")


# speedup vs baseline: 1.8482x; 1.8482x over previous
import functools

import jax
import jax.numpy as jnp
from jax import lax
from jax.experimental import pallas as pl
from jax.experimental.pallas import tpu as pltpu
from jax.experimental.pallas import tpu_sc as plsc

_CHUNK = 800


@functools.lru_cache(maxsize=None)
def _make_lookup(num_idx: int, vocab: int, dim: int):
    info = plsc.get_sparse_core_info()
    nc, ns = info.num_cores, info.num_subcores
    nw = nc * ns
    assert num_idx % (nw * _CHUNK) == 0
    b_per_w = num_idx // nw
    nchunks = b_per_w // _CHUNK
    mesh = plsc.VectorSubcoreMesh(core_axis_name="c", subcore_axis_name="s")

    @functools.partial(
        pl.kernel,
        out_type=jax.ShapeDtypeStruct((num_idx, dim), jnp.float32),
        mesh=mesh,
        scratch_types=[
            pltpu.VMEM((b_per_w,), jnp.int32),
            pltpu.VMEM((_CHUNK, dim), jnp.float32),
            pltpu.SemaphoreType.DMA,
        ],
        compiler_params=pltpu.CompilerParams(use_tc_tiling_on_sc=False),
    )
    def lookup(word_hbm, table_hbm, out_hbm, idx_v, rows_v, sem):
        wid = lax.axis_index("s") * nc + lax.axis_index("c")
        base = wid * b_per_w
        pltpu.sync_copy(word_hbm.at[pl.ds(base, b_per_w)], idx_v)

        def step(c, carry):
            off = c * _CHUNK
            pltpu.async_copy(
                table_hbm.at[idx_v.at[pl.ds(off, _CHUNK)]], rows_v, sem
            ).wait()
            pltpu.sync_copy(rows_v, out_hbm.at[pl.ds(base + off, _CHUNK)])
            return carry

        lax.fori_loop(0, nchunks, step, 0)

    return lookup


def kernel(word, table):
    batch, hist = word.shape
    vocab, dim = table.shape
    flat = word.reshape(batch * hist)
    out = _make_lookup(batch * hist, vocab, dim)(flat, table)
    return out.reshape(batch, hist, dim)


# speedup vs baseline: 1.8756x; 1.0148x over previous
import functools

import jax
import jax.numpy as jnp
from jax import lax
from jax.experimental import pallas as pl
from jax.experimental.pallas import tpu as pltpu
from jax.experimental.pallas import tpu_sc as plsc

_CHUNK = 800


@functools.lru_cache(maxsize=None)
def _make_lookup(num_idx: int, vocab: int, dim: int):
    info = plsc.get_sparse_core_info()
    nc, ns = info.num_cores, info.num_subcores
    nw = nc * ns
    assert num_idx % (nw * _CHUNK) == 0
    b_per_w = num_idx // nw
    nchunks = b_per_w // _CHUNK
    mesh = plsc.VectorSubcoreMesh(core_axis_name="c", subcore_axis_name="s")

    assert nchunks % 2 == 0 and nchunks >= 4

    @functools.partial(
        pl.kernel,
        out_type=jax.ShapeDtypeStruct((num_idx, dim), jnp.float32),
        mesh=mesh,
        scratch_types=[
            pltpu.VMEM((b_per_w,), jnp.int32),
            pltpu.VMEM((_CHUNK, dim), jnp.float32),
            pltpu.VMEM((_CHUNK, dim), jnp.float32),
            pltpu.SemaphoreType.DMA,
            pltpu.SemaphoreType.DMA,
            pltpu.SemaphoreType.DMA,
            pltpu.SemaphoreType.DMA,
        ],
        compiler_params=pltpu.CompilerParams(use_tc_tiling_on_sc=False),
    )
    def lookup(word_hbm, table_hbm, out_hbm, idx_v, rows0, rows1,
               g0, g1, o0, o1):
        wid = lax.axis_index("s") * nc + lax.axis_index("c")
        base = wid * b_per_w
        pltpu.sync_copy(word_hbm.at[pl.ds(base, b_per_w)], idx_v)
        bufs = (rows0, rows1)
        gsems = (g0, g1)
        osems = (o0, o1)

        def gather(c, buf, sem):
            return pltpu.make_async_copy(
                table_hbm.at[idx_v.at[pl.ds(c * _CHUNK, _CHUNK)]], buf, sem
            )

        def writeback(c, buf, sem):
            return pltpu.make_async_copy(
                buf, out_hbm.at[pl.ds(base + c * _CHUNK, _CHUNK)], sem
            )

        gather(0, rows0, g0).start()
        gather(1, rows1, g1).start()

        def pair(p, carry):
            for b in range(2):
                c = 2 * p + b
                gather(c, bufs[b], gsems[b]).wait()
                writeback(c, bufs[b], osems[b]).start()
                writeback(c, bufs[b], osems[b]).wait()
                gather(c + 2, bufs[b], gsems[b]).start()
            return carry

        lax.fori_loop(0, nchunks // 2 - 1, pair, 0)

        for b in range(2):
            c = nchunks - 2 + b
            gather(c, bufs[b], gsems[b]).wait()
            writeback(c, bufs[b], osems[b]).start()
        for b in range(2):
            c = nchunks - 2 + b
            writeback(c, bufs[b], osems[b]).wait()

    return lookup


def kernel(word, table):
    batch, hist = word.shape
    vocab, dim = table.shape
    flat = word.reshape(batch * hist)
    out = _make_lookup(batch * hist, vocab, dim)(flat, table)
    return out.reshape(batch, hist, dim)
